# baseline (speedup 1.0000x reference)
import jax
import jax.numpy as jnp
from jax.experimental import pallas as pl
from jax.experimental.pallas import tpu as pltpu

_OB, _OT = 0, 1
_PREGO, _PREB, _PRET, _PRESLOT = 2, 3, 4, 5
_POGO, _POB, _POT, _POSLOT = 6, 7, 8, 9
_WGO, _WSLOT = 10, 11
_NSPLIT = 4


def _mask_kernel(sched_ref, kn_ref, s_ref, mt_ref, emb_ref, out_ref,
                 islot0, islot1, obuf0, obuf1, keep_ref,
                 rsem0, rsem1, wsem0, wsem1):
    k = pl.program_id(0)
    nstep = pl.num_programs(0)
    n = s_ref.shape[1]
    bb, t_blk, d = obuf0.shape
    hb = bb // _NSPLIT

    @pl.when(k == 0)
    def _compute_keep():
        s_row = s_ref[...]
        i_row = jax.lax.broadcasted_iota(jnp.int32, (1, n), 1)
        valid = (i_row < kn_ref[0]).astype(jnp.int32)
        t_col = jax.lax.broadcasted_iota(jnp.int32, (n, 1), 0)
        hit = jnp.where(s_row == t_col, valid, 0)
        keep_ref[...] = jnp.max(hit, axis=1, keepdims=True)

    def _in_blk(b_idx, t_idx):
        return emb_ref.at[pl.ds(b_idx * bb, bb), pl.ds(t_idx * t_blk, t_blk), :]

    def _issue_read(go_row, b_row, t_row, slot_row, slot_ref, sem, slot_id):
        @pl.when(jnp.logical_and(sched_ref[go_row, k] == 1,
                                 sched_ref[slot_row, k] == slot_id))
        def _():
            pltpu.make_async_copy(
                _in_blk(sched_ref[b_row, k], sched_ref[t_row, k]),
                slot_ref, sem).start()

    def _write_op(step, obuf, sem, half):
        b_idx = sched_ref[_OB, step]
        t_idx = sched_ref[_OT, step]
        dst = out_ref.at[pl.ds(b_idx * bb + half * hb, hb),
                         pl.ds(t_idx * t_blk, t_blk), :]
        return pltpu.make_async_copy(obuf.at[pl.ds(half * hb, hb)], dst, sem)

    def _wait_write(step, obuf, sem):
        for h in range(_NSPLIT):
            _write_op(step, obuf, sem, h).wait()

    def _issue_write(step, obuf, sem):
        for h in range(_NSPLIT):
            _write_op(step, obuf, sem, h).start()

    _issue_read(_PREGO, _PREB, _PRET, _PRESLOT, islot0, rsem0, 0)
    _issue_read(_PREGO, _PREB, _PRET, _PRESLOT, islot1, rsem1, 1)

    mt_bcast = jnp.broadcast_to(mt_ref[0, 0][None, None, :], (bb, t_blk, d))
    km2 = jnp.maximum(k - 2, 0)
    parity = k % 2

    def _per_buf(obuf, wsem, pid):
        @pl.when(jnp.logical_and(k >= 2, parity == pid))
        def _wait_prev():
            _wait_write(km2, obuf, wsem)

        @pl.when(jnp.logical_and(parity == pid, sched_ref[_WGO, k] == 0))
        def _fill_masked():
            obuf[...] = mt_bcast

        def _fill_select(islot, rsem, slot_id):
            @pl.when(jnp.logical_and(
                parity == pid,
                jnp.logical_and(sched_ref[_WGO, k] == 1,
                                sched_ref[_WSLOT, k] == slot_id)))
            def _():
                pltpu.make_async_copy(
                    _in_blk(sched_ref[_OB, k], sched_ref[_OT, k]),
                    islot, rsem).wait()
                t0 = sched_ref[_OT, k] * t_blk
                keep_blk = keep_ref[pl.ds(t0, t_blk), :]
                obuf[...] = jnp.where(keep_blk[None, :, :] != 0,
                                      islot[...], mt_bcast)

        _fill_select(islot0, rsem0, 0)
        _fill_select(islot1, rsem1, 1)

        @pl.when(parity == pid)
        def _start_write():
            _issue_write(k, obuf, wsem)

    _per_buf(obuf0, wsem0, 0)
    _per_buf(obuf1, wsem1, 1)

    _issue_read(_POGO, _POB, _POT, _POSLOT, islot0, rsem0, 0)
    _issue_read(_POGO, _POB, _POT, _POSLOT, islot1, rsem1, 1)

    @pl.when(k == nstep - 1)
    def _drain():
        @pl.when(jnp.logical_and(k >= 1, parity == 0))
        def _():
            _wait_write(jnp.maximum(k - 1, 0), obuf1, wsem1)

        @pl.when(jnp.logical_and(k >= 1, parity == 1))
        def _():
            _wait_write(jnp.maximum(k - 1, 0), obuf0, wsem0)

        @pl.when(parity == 0)
        def _():
            _wait_write(k, obuf0, wsem0)

        @pl.when(parity == 1)
        def _():
            _wait_write(k, obuf1, wsem1)


def kernel(embeddings, mask_token, shuffled_indices, skip):
    B, N, D = embeddings.shape
    n = shuffled_indices.shape[0]
    T = 256 if n % 256 == 0 else n
    TB = n // T
    BB = 16 if B % 16 == 0 else (4 if B % 4 == 0 else 1)
    NB = B // BB
    NSTEP = NB * TB

    keep_n = jnp.asarray(n - skip, dtype=jnp.int32).reshape(1)
    s2d = shuffled_indices.astype(jnp.int32).reshape(1, n)

    idx = jnp.arange(n, dtype=jnp.int32)
    in_blk = shuffled_indices.astype(jnp.int32) // T
    is_kept = (idx < keep_n[0]).astype(jnp.int32)
    counts = jnp.sum(
        jnp.where(in_blk[:, None] == jnp.arange(TB, dtype=jnp.int32)[None, :],
                  is_kept[:, None], 0),
        axis=0)
    need = jnp.tile((counts > 0).astype(jnp.int32), NB)

    m = NSTEP - jnp.sum(need)
    nneed = jnp.sum(need)
    L = jnp.minimum(m, 2)
    midx = jnp.cumsum(1 - need) - 1
    nidx = jnp.cumsum(need) - 1
    key = jnp.where(need == 1, L + nidx,
                    jnp.where(midx < L, midx, nneed + midx))
    order = jnp.argsort(key)
    ob, ot = order // TB, order % TB
    j = jnp.arange(NSTEP)
    validj = j < nneed
    jpos = jnp.clip(L + j, 0, NSTEP - 1)
    jb, jt = ob[jpos], ot[jpos]

    zeros = jnp.zeros((NSTEP,), jnp.int32)
    pre_idx = jnp.where(jnp.logical_and(validj, j < 2), j, NSTEP)
    prego = zeros.at[pre_idx].set(1, mode="drop")
    preb = zeros.at[pre_idx].set(jb, mode="drop")
    pret = zeros.at[pre_idx].set(jt, mode="drop")
    preslot = zeros.at[pre_idx].set(j % 2, mode="drop")
    post_idx = jnp.where(jnp.logical_and(validj, j >= 2), L + j - 2, NSTEP)
    pogo = zeros.at[post_idx].set(1, mode="drop")
    pob = zeros.at[post_idx].set(jb, mode="drop")
    pot = zeros.at[post_idx].set(jt, mode="drop")
    poslot = zeros.at[post_idx].set(j % 2, mode="drop")
    wgo = jnp.take(need, order)
    wslot = jnp.where(wgo == 1, (j - L) % 2, 0)

    sched = jnp.stack(
        [ob, ot, prego, preb, pret, preslot, pogo, pob, pot, poslot,
         wgo, wslot]).astype(jnp.int32)

    grid_spec = pltpu.PrefetchScalarGridSpec(
        num_scalar_prefetch=2,
        grid=(NSTEP,),
        in_specs=[
            pl.BlockSpec((1, n), lambda k, sc, kn: (0, 0)),
            pl.BlockSpec((1, 1, D), lambda k, sc, kn: (0, 0, 0)),
            pl.BlockSpec(memory_space=pl.ANY),
        ],
        out_specs=pl.BlockSpec(memory_space=pl.ANY),
        scratch_shapes=[
            pltpu.VMEM((BB, T, D), jnp.float32),
            pltpu.VMEM((BB, T, D), jnp.float32),
            pltpu.VMEM((BB, T, D), jnp.float32),
            pltpu.VMEM((BB, T, D), jnp.float32),
            pltpu.VMEM((N, 1), jnp.int32),
            pltpu.SemaphoreType.DMA,
            pltpu.SemaphoreType.DMA,
            pltpu.SemaphoreType.DMA,
            pltpu.SemaphoreType.DMA,
        ],
    )

    return pl.pallas_call(
        _mask_kernel,
        grid_spec=grid_spec,
        out_shape=jax.ShapeDtypeStruct((B, N, D), embeddings.dtype),
    )(sched, keep_n, s2d, mask_token, embeddings)


# speedup vs baseline: 1.1344x; 1.1344x over previous
import jax
import jax.numpy as jnp
from jax.experimental import pallas as pl
from jax.experimental.pallas import tpu as pltpu


def _mask_kernel(bm_ref, kn_ref, s_ref, emb_ref, mt_ref, out_ref, keep_ref):
    b = pl.program_id(0)
    tb = pl.program_id(1)
    n = keep_ref.shape[0]

    @pl.when(jnp.logical_and(b == 0, tb == 0))
    def _compute_keep():
        s_row = s_ref[...]
        i_row = jax.lax.broadcasted_iota(jnp.int32, (1, n), 1)
        valid = (i_row < kn_ref[0]).astype(jnp.int32)
        t_col = jax.lax.broadcasted_iota(jnp.int32, (n, 1), 0)
        hit = jnp.where(s_row == t_col, valid, 0)
        keep_ref[...] = jnp.max(hit, axis=1, keepdims=True)

    t_blk = out_ref.shape[1]
    keep_blk = keep_ref[pl.ds(tb * t_blk, t_blk), :]
    out_ref[...] = jnp.where(keep_blk[None, :, :] != 0, emb_ref[...],
                             mt_ref[0, 0][None, None, :])


def kernel(embeddings, mask_token, shuffled_indices, skip):
    B, N, D = embeddings.shape
    n = shuffled_indices.shape[0]
    T = 256 if n % 256 == 0 else n
    TB = n // T
    BB = 16 if B % 16 == 0 else (4 if B % 4 == 0 else 1)

    keep_n = jnp.asarray(n - skip, dtype=jnp.int32).reshape(1)
    s2d = shuffled_indices.astype(jnp.int32).reshape(1, n)

    idx = jnp.arange(n, dtype=jnp.int32)
    in_blk = shuffled_indices.astype(jnp.int32) // T
    is_kept = (idx < keep_n[0]).astype(jnp.int32)
    counts = jnp.sum(
        jnp.where(in_blk[:, None] == jnp.arange(TB, dtype=jnp.int32)[None, :],
                  is_kept[:, None], 0),
        axis=0)
    bm = jax.lax.cummax(jnp.where(counts > 0, jnp.arange(TB, dtype=jnp.int32), 0))

    grid_spec = pltpu.PrefetchScalarGridSpec(
        num_scalar_prefetch=2,
        grid=(B // BB, TB),
        in_specs=[
            pl.BlockSpec((1, n), lambda b, tb, bm, kn: (0, 0)),
            pl.BlockSpec((BB, T, D), lambda b, tb, bm, kn: (b, bm[tb], 0)),
            pl.BlockSpec((1, 1, D), lambda b, tb, bm, kn: (0, 0, 0)),
        ],
        out_specs=pl.BlockSpec((BB, T, D), lambda b, tb, bm, kn: (b, tb, 0)),
        scratch_shapes=[pltpu.VMEM((n, 1), jnp.int32)],
    )

    return pl.pallas_call(
        _mask_kernel,
        grid_spec=grid_spec,
        out_shape=jax.ShapeDtypeStruct((B, N, D), embeddings.dtype),
    )(bm, keep_n, s2d, embeddings, mask_token)
